# baseline (speedup 1.0000x reference)
import jax
import jax.numpy as jnp
from jax.experimental import pallas as pl
from jax.experimental.pallas import tpu as pltpu


_NCHUNKS = 32


def _copy_body(w_ref, o_ref, sems):
    rows = w_ref.shape[0]
    r = rows // _NCHUNKS
    copies = [
        pltpu.make_async_copy(
            w_ref.at[pl.ds(i * r, r), :],
            o_ref.at[0, pl.ds(i * r, r), :],
            sems.at[i],
        )
        for i in range(_NCHUNKS)
    ]
    for c in copies:
        c.start()
    for c in copies:
        c.wait()


def kernel(x, emb_weight):
    seq_len = x.shape[1]
    out = pl.pallas_call(
        _copy_body,
        out_shape=jax.ShapeDtypeStruct(
            (1, seq_len, emb_weight.shape[1]), emb_weight.dtype
        ),
        in_specs=[pl.BlockSpec(memory_space=pl.ANY)],
        out_specs=pl.BlockSpec(memory_space=pl.ANY),
        scratch_shapes=[pltpu.SemaphoreType.DMA((_NCHUNKS,))],
    )(emb_weight)
    return out


# speedup vs baseline: 41.1676x; 41.1676x over previous
import jax
import jax.numpy as jnp
from jax.experimental import pallas as pl
from jax.experimental.pallas import tpu as pltpu


_BLOCK_ROWS = 512


def _copy_body(w_ref, o_ref):
    o_ref[...] = w_ref[...][None]


def kernel(x, emb_weight):
    seq_len = x.shape[1]
    dim = emb_weight.shape[1]
    grid = (seq_len // _BLOCK_ROWS,)
    out = pl.pallas_call(
        _copy_body,
        grid=grid,
        out_shape=jax.ShapeDtypeStruct((1, seq_len, dim), emb_weight.dtype),
        in_specs=[pl.BlockSpec((_BLOCK_ROWS, dim), lambda i: (i, 0))],
        out_specs=pl.BlockSpec((1, _BLOCK_ROWS, dim), lambda i: (0, i, 0)),
    )(emb_weight)
    return out


# speedup vs baseline: 45.1014x; 1.0956x over previous
import jax
import jax.numpy as jnp
from jax.experimental import pallas as pl
from jax.experimental.pallas import tpu as pltpu


_BLOCK_ROWS = 1024


def _copy_body(w_ref, o_ref):
    o_ref[...] = w_ref[...][None]


def kernel(x, emb_weight):
    seq_len = x.shape[1]
    dim = emb_weight.shape[1]
    grid = (seq_len // _BLOCK_ROWS,)
    out = pl.pallas_call(
        _copy_body,
        grid=grid,
        out_shape=jax.ShapeDtypeStruct((1, seq_len, dim), emb_weight.dtype),
        in_specs=[pl.BlockSpec((_BLOCK_ROWS, dim), lambda i: (i, 0))],
        out_specs=pl.BlockSpec((1, _BLOCK_ROWS, dim), lambda i: (0, i, 0)),
    )(emb_weight)
    return out


# speedup vs baseline: 48.9807x; 1.0860x over previous
import jax
import jax.numpy as jnp
from jax.experimental import pallas as pl
from jax.experimental.pallas import tpu as pltpu


_BLOCK_ROWS = 2048


def _copy_body(w_ref, o_ref):
    o_ref[...] = w_ref[...][None]


def kernel(x, emb_weight):
    seq_len = x.shape[1]
    dim = emb_weight.shape[1]
    grid = (seq_len // _BLOCK_ROWS,)
    out = pl.pallas_call(
        _copy_body,
        grid=grid,
        out_shape=jax.ShapeDtypeStruct((1, seq_len, dim), emb_weight.dtype),
        in_specs=[pl.BlockSpec((_BLOCK_ROWS, dim), lambda i: (i, 0))],
        out_specs=pl.BlockSpec((1, _BLOCK_ROWS, dim), lambda i: (0, i, 0)),
    )(emb_weight)
    return out
